# Initial kernel scaffold; baseline (speedup 1.0000x reference)
import jax
import jax.numpy as jnp
from jax.experimental import pallas as pl


def kernel(atom_features, bond_features, pair_indices, kernel, bias):
    raise NotImplementedError("write your pallas kernel here")


# speedup vs baseline: 4.1217x; 4.1217x over previous
import functools

import jax
import jax.numpy as jnp
from jax import lax
from jax.experimental import pallas as pl
from jax.experimental.pallas import tpu as pltpu
from jax.experimental.pallas import tpu_sc as plsc

NC = 2
NS = 16
CH = 112


def _gather_kernel(e_pad, d, n_chunks, e_per_tile):
    mesh = plsc.VectorSubcoreMesh(
        core_axis_name="c", subcore_axis_name="s", num_cores=NC, num_subcores=NS
    )

    @functools.partial(
        pl.kernel,
        out_type=jax.ShapeDtypeStruct((e_pad, d), jnp.float32),
        mesh=mesh,
        scratch_types=[
            pltpu.VMEM((CH,), jnp.int32),
            pltpu.VMEM((CH, d), jnp.float32),
            pltpu.SemaphoreType.DMA,
        ],
        compiler_params=pltpu.CompilerParams(use_tc_tiling_on_sc=False),
    )
    def gather_k(atoms_hbm, src_hbm, neigh_hbm, idx_v, rows_v, sem):
        wid = lax.axis_index("s") * NC + lax.axis_index("c")
        tbase = wid * e_per_tile

        def body(ci, carry):
            base = tbase + ci * CH
            pltpu.sync_copy(src_hbm.at[pl.ds(base, CH)], idx_v)
            pltpu.async_copy(atoms_hbm.at[idx_v], rows_v, sem).wait()
            pltpu.sync_copy(rows_v, neigh_hbm.at[pl.ds(base, CH)])
            return carry

        lax.fori_loop(0, n_chunks, body, 0)

    return gather_k


def _scatter_kernel(e_pad, d, na_half, s_rows, rows_per_tile, n_edge_chunks):
    mesh = plsc.VectorSubcoreMesh(
        core_axis_name="c", subcore_axis_name="s", num_cores=NC, num_subcores=NS
    )
    n_row_chunks = rows_per_tile // CH
    e_per_tile = e_pad // NS

    @functools.partial(
        pl.kernel,
        out_type=jax.ShapeDtypeStruct((NC * s_rows, d), jnp.float32),
        mesh=mesh,
        scratch_types=[
            pltpu.VMEM((CH,), jnp.int32),
            pltpu.VMEM((CH, d), jnp.float32),
            pltpu.VMEM_SHARED((s_rows, d), jnp.float32),
        ],
        compiler_params=pltpu.CompilerParams(use_tc_tiling_on_sc=False),
    )
    def scatter_k(msgs_hbm, dst_hbm, zrows_hbm, out_hbm, idx_v, rows_v, shared):
        c = lax.axis_index("c")
        s = lax.axis_index("s")
        core_base = c * na_half
        tile_rows0 = s * rows_per_tile

        pltpu.sync_copy(zrows_hbm, rows_v)

        def zbody(i, carry):
            pltpu.sync_copy(rows_v, shared.at[pl.ds(tile_rows0 + i * CH, CH)])
            return carry

        lax.fori_loop(0, n_row_chunks, zbody, 0)
        plsc.subcore_barrier()

        def sbody(ci, carry):
            ebase = s * e_per_tile + ci * CH
            pltpu.sync_copy(dst_hbm.at[pl.ds(ebase, CH)], idx_v)
            for j in range(CH // 16):
                v = idx_v[pl.ds(j * 16, 16)]
                local = v - core_base
                ok = (local >= 0) & (local < na_half)
                idx_v[pl.ds(j * 16, 16)] = jnp.where(ok, local, na_half)
            pltpu.sync_copy(msgs_hbm.at[pl.ds(ebase, CH)], rows_v)
            pltpu.sync_copy(rows_v, shared.at[idx_v], add=True)
            return carry

        lax.fori_loop(0, n_edge_chunks, sbody, 0)
        plsc.subcore_barrier()

        def obody(i, carry):
            r = tile_rows0 + i * CH
            pltpu.sync_copy(shared.at[pl.ds(r, CH)], rows_v)
            pltpu.sync_copy(rows_v, out_hbm.at[pl.ds(c * s_rows + r, CH)])
            return carry

        lax.fori_loop(0, n_row_chunks, obody, 0)

    return scatter_k


def _tc_messages(neigh, bond, w2, r_mat, bt, blk):
    e_pad, d = neigh.shape
    bd = bond.shape[1]

    def body(neigh_ref, bond_ref, w2_ref, r_ref, bt_ref, out_ref):
        nb = neigh_ref[...]
        bexp = jnp.dot(bond_ref[...], r_ref[...], preferred_element_type=jnp.float32)
        ntile = jnp.concatenate([nb] * bd, axis=1)
        msg = jnp.dot(bexp * ntile, w2_ref[...], preferred_element_type=jnp.float32)
        msg = msg + jnp.dot(nb, bt_ref[...], preferred_element_type=jnp.float32)
        out_ref[...] = msg

    return pl.pallas_call(
        body,
        grid=(e_pad // blk,),
        in_specs=[
            pl.BlockSpec((blk, d), lambda i: (i, 0)),
            pl.BlockSpec((blk, bd), lambda i: (i, 0)),
            pl.BlockSpec((bd * d, d), lambda i: (0, 0)),
            pl.BlockSpec((bd, bd * d), lambda i: (0, 0)),
            pl.BlockSpec((d, d), lambda i: (0, 0)),
        ],
        out_specs=pl.BlockSpec((blk, d), lambda i: (i, 0)),
        out_shape=jax.ShapeDtypeStruct((e_pad, d), jnp.float32),
    )(neigh, bond, w2, r_mat, bt)


def kernel(atom_features, bond_features, pair_indices, kernel, bias):
    n_atoms, d = atom_features.shape
    e = bond_features.shape[0]
    bd = bond_features.shape[1]
    assert n_atoms % 2 == 0

    tile_quant = NC * NS * CH
    e_pad = ((e + tile_quant - 1) // tile_quant) * tile_quant
    e_per_tile = e_pad // (NC * NS)
    n_chunks = e_per_tile // CH

    pi = pair_indices.astype(jnp.int32)
    src = jnp.concatenate([pi[:, 1], jnp.zeros((e_pad - e,), jnp.int32)])
    dst = jnp.concatenate([pi[:, 0], jnp.full((e_pad - e,), -1, jnp.int32)])
    bond_pad = jnp.concatenate(
        [bond_features, jnp.zeros((e_pad - e, bd), jnp.float32)], axis=0
    )

    w2 = kernel.reshape(bd, d, d).transpose(0, 2, 1).reshape(bd * d, d)
    bt = bias.reshape(d, d).T
    r_mat = jnp.repeat(jnp.eye(bd, dtype=jnp.float32), d, axis=1)

    neigh = _gather_kernel(e_pad, d, n_chunks, e_per_tile)(atom_features, src)

    messages = _tc_messages(neigh, bond_pad, w2, r_mat, bt, blk=512)

    na_half = n_atoms // 2
    rows_per_tile = ((na_half + NS * CH) // (NS * CH)) * CH
    s_rows = NS * rows_per_tile
    n_edge_chunks = e_pad // (NS * CH)
    zrows = jnp.zeros((CH, d), jnp.float32)
    out_pad = _scatter_kernel(e_pad, d, na_half, s_rows, rows_per_tile, n_edge_chunks)(
        messages, dst, zrows
    )
    return jnp.concatenate(
        [out_pad[:na_half], out_pad[s_rows : s_rows + na_half]], axis=0
    )


# speedup vs baseline: 4.8220x; 1.1699x over previous
import functools

import jax
import jax.numpy as jnp
from jax import lax
from jax.experimental import pallas as pl
from jax.experimental.pallas import tpu as pltpu
from jax.experimental.pallas import tpu_sc as plsc

NC = 2
NS = 16
CH = 112
GC = 7
SG = 1


def _gather_kernel(e_pad, d, n_chunks, e_per_tile):
    mesh = plsc.VectorSubcoreMesh(
        core_axis_name="c", subcore_axis_name="s", num_cores=NC, num_subcores=NS
    )
    n_groups = n_chunks // GC
    grp_rows = GC * CH

    @functools.partial(
        pl.kernel,
        out_type=jax.ShapeDtypeStruct((e_pad, d), jnp.float32),
        mesh=mesh,
        scratch_types=[
            pltpu.VMEM((n_chunks, CH), jnp.int32),
            pltpu.VMEM((grp_rows, d), jnp.float32),
            pltpu.VMEM((grp_rows, d), jnp.float32),
            pltpu.SemaphoreType.DMA,
            pltpu.SemaphoreType.DMA,
        ],
        compiler_params=pltpu.CompilerParams(use_tc_tiling_on_sc=False),
    )
    def gather_k(atoms_hbm, src_hbm, neigh_hbm, idx2, rows0, rows1, sem0, sem1):
        wid = lax.axis_index("s") * NC + lax.axis_index("c")
        tbase = wid * e_per_tile

        pltpu.sync_copy(src_hbm.at[wid], idx2)

        rows = (rows0, rows1)
        sems = (sem0, sem1)
        handles = []
        for g in range(n_groups):
            hg = []
            for k in range(GC):
                ci = g * GC + k
                hg.append(
                    pltpu.async_copy(
                        atoms_hbm.at[idx2.at[ci]],
                        rows[g % 2].at[pl.ds(k * CH, CH)],
                        sems[g % 2],
                    )
                )
            handles.append(hg)
        for g in range(n_groups):
            for h in handles[g]:
                h.wait()
            pltpu.sync_copy(
                rows[g % 2], neigh_hbm.at[pl.ds(tbase + g * grp_rows, grp_rows)]
            )

    return gather_k


def _scatter_kernel(e_pad, d, n_atoms, na_half, s_rows, rows_per_tile, n_ec):
    mesh = plsc.VectorSubcoreMesh(
        core_axis_name="c", subcore_axis_name="s", num_cores=NC, num_subcores=NS
    )
    e_per_tile = e_pad // NS
    n_sg = n_ec // SG
    grp_rows = SG * CH
    full_tiles = na_half // rows_per_tile
    rem_rows = na_half - full_tiles * rows_per_tile

    @functools.partial(
        pl.kernel,
        out_type=jax.ShapeDtypeStruct((n_atoms, d), jnp.float32),
        mesh=mesh,
        scratch_types=[
            pltpu.VMEM((n_ec, CH), jnp.int32),
            pltpu.VMEM((grp_rows, d), jnp.float32),
            pltpu.VMEM((grp_rows, d), jnp.float32),
            pltpu.VMEM_SHARED((s_rows, d), jnp.float32),
            pltpu.SemaphoreType.DMA,
            pltpu.SemaphoreType.DMA,
        ],
        compiler_params=pltpu.CompilerParams(use_tc_tiling_on_sc=False),
    )
    def scatter_k(msgs_hbm, dst_hbm, zeros_hbm, out_hbm, idxs, rows0, rows1,
                  shared, sem0, sem1):
        c = lax.axis_index("c")
        s = lax.axis_index("s")
        core_base = c * na_half
        tile_rows0 = s * rows_per_tile

        pltpu.sync_copy(zeros_hbm, shared.at[pl.ds(tile_rows0, rows_per_tile)])
        pltpu.sync_copy(dst_hbm.at[s], idxs)
        plsc.subcore_barrier()

        rows = (rows0, rows1)
        sems = (sem0, sem1)

        def remap_group(g):
            for k in range(SG):
                ci = g * SG + k
                for j in range(CH // 16):
                    v = idxs[ci, pl.ds(j * 16, 16)]
                    local = v - core_base
                    ok = (local >= 0) & (local < na_half)
                    idxs[ci, pl.ds(j * 16, 16)] = jnp.where(ok, local, na_half)

        def load_group(g, b):
            ebase = s * e_per_tile + g * grp_rows
            return pltpu.async_copy(
                msgs_hbm.at[pl.ds(ebase, grp_rows)], rows[b], sems[b]
            )

        pending = load_group(0, 0)
        remap_group(0)
        for g in range(n_sg):
            b = g % 2
            if g + 1 < n_sg:
                nxt = load_group(g + 1, 1 - b)
            pending.wait()
            for k in range(SG):
                pltpu.sync_copy(
                    rows[b].at[pl.ds(k * CH, CH)],
                    shared.at[idxs.at[g * SG + k]],
                    add=True,
                )
            if g + 1 < n_sg:
                remap_group(g + 1)
                pending = nxt
        plsc.subcore_barrier()

        if full_tiles > 0:
            @pl.when(s < full_tiles)
            def _():
                pltpu.sync_copy(
                    shared.at[pl.ds(tile_rows0, rows_per_tile)],
                    out_hbm.at[pl.ds(core_base + tile_rows0, rows_per_tile)],
                )
        if rem_rows > 0:
            @pl.when(s == full_tiles)
            def _():
                pltpu.sync_copy(
                    shared.at[pl.ds(tile_rows0, rem_rows)],
                    out_hbm.at[pl.ds(core_base + tile_rows0, rem_rows)],
                )

    return scatter_k


def _tc_messages(neigh, bond, w2, r_mat, bt, blk):
    e_pad, d = neigh.shape
    bd = bond.shape[1]

    def body(neigh_ref, bond_ref, w2_ref, r_ref, bt_ref, out_ref):
        nb = neigh_ref[...]
        bexp = jnp.dot(bond_ref[...], r_ref[...], preferred_element_type=jnp.float32)
        ntile = jnp.concatenate([nb] * bd, axis=1)
        msg = jnp.dot(bexp * ntile, w2_ref[...], preferred_element_type=jnp.float32)
        msg = msg + jnp.dot(nb, bt_ref[...], preferred_element_type=jnp.float32)
        out_ref[...] = msg

    return pl.pallas_call(
        body,
        grid=(e_pad // blk,),
        in_specs=[
            pl.BlockSpec((blk, d), lambda i: (i, 0)),
            pl.BlockSpec((blk, bd), lambda i: (i, 0)),
            pl.BlockSpec((bd * d, d), lambda i: (0, 0)),
            pl.BlockSpec((bd, bd * d), lambda i: (0, 0)),
            pl.BlockSpec((d, d), lambda i: (0, 0)),
        ],
        out_specs=pl.BlockSpec((blk, d), lambda i: (i, 0)),
        out_shape=jax.ShapeDtypeStruct((e_pad, d), jnp.float32),
    )(neigh, bond, w2, r_mat, bt)


def kernel(atom_features, bond_features, pair_indices, kernel, bias):
    n_atoms, d = atom_features.shape
    e = bond_features.shape[0]
    bd = bond_features.shape[1]
    assert n_atoms % 2 == 0

    tile_quant = NC * NS * CH
    e_pad = ((e + tile_quant - 1) // tile_quant) * tile_quant
    e_per_tile = e_pad // (NC * NS)
    n_chunks = e_per_tile // CH

    pi = pair_indices.astype(jnp.int32)
    src = jnp.concatenate([pi[:, 1], jnp.zeros((e_pad - e,), jnp.int32)])
    dst = jnp.concatenate([pi[:, 0], jnp.full((e_pad - e,), -1, jnp.int32)])
    bond_pad = jnp.concatenate(
        [bond_features, jnp.zeros((e_pad - e, bd), jnp.float32)], axis=0
    )

    w2 = kernel.reshape(bd, d, d).transpose(0, 2, 1).reshape(bd * d, d)
    bt = bias.reshape(d, d).T
    r_mat = jnp.repeat(jnp.eye(bd, dtype=jnp.float32), d, axis=1)

    src3 = src.reshape(NC * NS, n_chunks, CH)
    neigh = _gather_kernel(e_pad, d, n_chunks, e_per_tile)(atom_features, src3)

    messages = _tc_messages(neigh, bond_pad, w2, r_mat, bt, blk=512)

    na_half = n_atoms // 2
    rows_per_tile = ((na_half + NS * CH) // (NS * CH)) * CH
    s_rows = NS * rows_per_tile
    n_ec = e_pad // (NS * CH)
    dst3 = dst.reshape(NS, n_ec, CH)
    zeros = jnp.zeros((rows_per_tile, d), jnp.float32)
    return _scatter_kernel(e_pad, d, n_atoms, na_half, s_rows, rows_per_tile, n_ec)(
        messages, dst3, zeros
    )
